# baseline (speedup 1.0000x reference)
import jax
import jax.numpy as jnp
from jax.experimental import pallas as pl
from jax.experimental.pallas import tpu as pltpu

B = 16
K = 5
V = 8192
H = 1024
E = 512
EE = 64
NEMO = 7
T = 8
SOS = 1
EOS = 2

RES = 2048
SW = 512
S = (V - RES) // SW
BK = B * K
FLOOR = -1.0e30
FLOOR2 = -2.0e30
BIGIDX = 2 ** 30

F32 = jnp.float32


def _decode_kernel(enc_ref, emoid_ref, tok_ref, emoemb_ref, wih_ref,
                   whh_ref, bih_ref, bhh_ref, woutres_ref, boutres_ref,
                   wout_ref, bout_ref,
                   logp_ref, hid_out_ref, vocab_ref, beam_ref, score_ref,
                   hid_s, hnew_s, seq_s, ids_s, xcat_s, cvals_s, cidx_s,
                   selbeam_s, selvocab_s, selscore_s):
    t = pl.program_id(0)
    s = pl.program_id(1)

    @pl.when((t == 0) & (s == 0))
    def _init():
        enc = enc_ref[...]
        for k in range(K):
            hid_s[16 * k:16 * (k + 1), :] = enc
        seq_s[0:16, :] = jnp.zeros((16, 1), F32)
        seq_s[16:BK, :] = jnp.full((BK - 16, 1), -jnp.inf, F32)
        ids_s[...] = jnp.full((BK, 1), SOS, jnp.int32)
        lane8 = jax.lax.broadcasted_iota(jnp.int32, (BK, 8), 1)
        emo_oh = (lane8 == emoid_ref[...]).astype(F32)
        xcat_s[:, E:] = jnp.dot(emo_oh, emoemb_ref[...],
                                preferred_element_type=F32,
                                precision=jax.lax.Precision.HIGHEST)

    @pl.when(s == 0)
    def _gru():
        lanes_v = jax.lax.broadcasted_iota(jnp.int32, (BK, V), 1)
        onehot = (lanes_v == ids_s[...]).astype(F32)
        xcat_s[:, :E] = jnp.dot(onehot, tok_ref[...],
                                preferred_element_type=F32,
                                precision=jax.lax.Precision.HIGHEST)
        h = hid_s[...]
        gx = (jnp.dot(xcat_s[...], wih_ref[...], preferred_element_type=F32)
              + bih_ref[...])
        gh = (jnp.dot(h, whh_ref[...], preferred_element_type=F32)
              + bhh_ref[...])
        r = jax.nn.sigmoid(gx[:, :H] + gh[:, :H])
        z = jax.nn.sigmoid(gx[:, H:2 * H] + gh[:, H:2 * H])
        n = jnp.tanh(gx[:, 2 * H:] + r * gh[:, 2 * H:])
        hnew_s[...] = (1.0 - z) * n + z * h
        logp_ref[0, :, 0:RES] = (
            jnp.dot(hnew_s[...], woutres_ref[...],
                    preferred_element_type=F32) + boutres_ref[...])

    logp_ref[0, :, pl.ds(RES + s * SW, SW)] = (
        jnp.dot(hnew_s[...], wout_ref[...], preferred_element_type=F32)
        + bout_ref[...])

    @pl.when(s == S - 1)
    def _step():
        m = jnp.max(logp_ref[0, :, :], axis=1, keepdims=True)
        shifted = logp_ref[0, :, :] - m
        e = jnp.exp(shifted)
        parts = [e[:, c * 128:(c + 1) * 128] for c in range(V // 128)]
        while len(parts) > 1:
            parts = [parts[2 * i] + parts[2 * i + 1]
                     for i in range(len(parts) // 2)]
        acc = parts[0]
        w0 = 128
        while w0 > 1:
            w0 //= 2
            acc = acc[:, :w0] + acc[:, w0:2 * w0]
        logp_ref[0, :, :] = shifted - jnp.log(acc)

        cvals_s[...] = jnp.full((B, 32), FLOOR2, F32)
        cidx_s[...] = jnp.full((B, 32), BIGIDX, jnp.int32)
        lanes16 = jax.lax.broadcasted_iota(jnp.int32, (B, V), 1)
        for k in range(K):
            w = jnp.maximum(seq_s[16 * k:16 * (k + 1), :]
                            + logp_ref[0, 16 * k:16 * (k + 1), :], FLOOR)
            for j in range(K):
                mv = jnp.max(w, axis=1, keepdims=True)
                c = jnp.min(jnp.where(w == mv, lanes16, V),
                            axis=1, keepdims=True)
                col = k * K + j
                cvals_s[:, col:col + 1] = mv
                cidx_s[:, col:col + 1] = k * V + c
                w = jnp.where(lanes16 == c, FLOOR2, w)

        vals = cvals_s[...]
        idxs = cidx_s[...]
        vocab_ref[0, :, :] = jnp.zeros((B, 8), jnp.int32)
        beam_ref[0, :, :] = jnp.zeros((B, 8), jnp.int32)
        score_ref[0, :, :] = jnp.zeros((B, 8), F32)
        for j in range(K):
            mv = jnp.max(vals, axis=1, keepdims=True)
            fsel = jnp.min(jnp.where(vals == mv, idxs, BIGIDX),
                           axis=1, keepdims=True)
            vj = jnp.where(mv == FLOOR, -jnp.inf, mv)
            vocab = jnp.bitwise_and(fsel, V - 1)
            beamk = jnp.right_shift(fsel, 13)
            selvocab_s[16 * j:16 * (j + 1), :] = vocab
            selbeam_s[16 * j:16 * (j + 1), :] = beamk
            selscore_s[16 * j:16 * (j + 1), :] = vj
            vocab_ref[0, :, j:j + 1] = vocab
            beam_ref[0, :, j:j + 1] = beamk
            score_ref[0, :, j:j + 1] = vj
            vals = jnp.where(idxs == fsel, FLOOR2, vals)

        hn = [hnew_s[16 * k:16 * (k + 1), :] for k in range(K)]
        for j in range(K):
            bsel = selbeam_s[16 * j:16 * (j + 1), :]
            blk = hn[K - 1]
            for k in range(K - 2, -1, -1):
                blk = jnp.where(bsel == k, hn[k], blk)
            hid_s[16 * j:16 * (j + 1), :] = blk
            hid_out_ref[16 * j:16 * (j + 1), :] = blk

        newids = selvocab_s[...]
        seq_s[...] = jnp.where(newids == EOS, -jnp.inf, selscore_s[...])
        ids_s[...] = newids


@jax.jit
def kernel(encoder_hidden, emotion_inputs, tok_embed, emo_embed, W_ih, W_hh,
           b_ih, b_hh, W_out, b_out):
    enc = encoder_hidden[0]
    emoid = jnp.tile(emotion_inputs[:, 0].astype(jnp.int32), K)[:, None]
    emoemb8 = jnp.zeros((8, EE), F32).at[:NEMO].set(emo_embed)
    grid = (T, S)
    const2d = lambda t, s: (0, 0)
    out_shapes = (
        jax.ShapeDtypeStruct((T, BK, V), F32),
        jax.ShapeDtypeStruct((BK, H), F32),
        jax.ShapeDtypeStruct((T, B, 8), jnp.int32),
        jax.ShapeDtypeStruct((T, B, 8), jnp.int32),
        jax.ShapeDtypeStruct((T, B, 8), F32),
    )
    out_specs = (
        pl.BlockSpec((1, BK, V), lambda t, s: (t, 0, 0)),
        pl.BlockSpec((BK, H), const2d),
        pl.BlockSpec((1, B, 8), lambda t, s: (t, 0, 0)),
        pl.BlockSpec((1, B, 8), lambda t, s: (t, 0, 0)),
        pl.BlockSpec((1, B, 8), lambda t, s: (t, 0, 0)),
    )
    in_specs = [
        pl.BlockSpec((B, H), const2d),
        pl.BlockSpec((BK, 1), const2d),
        pl.BlockSpec((V, E), const2d),
        pl.BlockSpec((8, EE), const2d),
        pl.BlockSpec((E + EE, 3 * H), const2d),
        pl.BlockSpec((H, 3 * H), const2d),
        pl.BlockSpec((1, 3 * H), const2d),
        pl.BlockSpec((1, 3 * H), const2d),
        pl.BlockSpec((H, RES), const2d),
        pl.BlockSpec((1, RES), const2d),
        pl.BlockSpec((H, SW), lambda t, s: (0, s + RES // SW)),
        pl.BlockSpec((1, SW), lambda t, s: (0, s + RES // SW)),
    ]
    scratch = [
        pltpu.VMEM((BK, H), F32),
        pltpu.VMEM((BK, H), F32),
        pltpu.VMEM((BK, 1), F32),
        pltpu.VMEM((BK, 1), jnp.int32),
        pltpu.VMEM((BK, E + EE), F32),
        pltpu.VMEM((B, 32), F32),
        pltpu.VMEM((B, 32), jnp.int32),
        pltpu.VMEM((BK, 1), jnp.int32),
        pltpu.VMEM((BK, 1), jnp.int32),
        pltpu.VMEM((BK, 1), F32),
    ]

    logp_bm, hid_bm, vocab5, beam5, score5 = pl.pallas_call(
        _decode_kernel,
        grid=grid,
        in_specs=in_specs,
        out_specs=out_specs,
        out_shape=out_shapes,
        scratch_shapes=scratch,
    )(enc, emoid, tok_embed, emoemb8, W_ih, W_hh,
      b_ih[None, :], b_hh[None, :], W_out[:, :RES], b_out[None, :RES],
      W_out, b_out[None, :])

    logp = logp_bm.reshape(T, K, B, V).transpose(0, 2, 1, 3).reshape(T, BK, V)
    hidden = hid_bm.reshape(K, B, H).transpose(1, 0, 2).reshape(BK, H)[None]
    symbols = vocab5[:, :, :K].reshape(T, BK)
    preds = (beam5[:, :, :K] +
             (jnp.arange(B, dtype=jnp.int32) * K)[None, :, None]
             ).reshape(T, BK)
    scores = score5[:, :, :K].reshape(T, BK)
    return logp, hidden, symbols, preds, scores


# speedup vs baseline: 1.0778x; 1.0778x over previous
import jax
import jax.numpy as jnp
from jax.experimental import pallas as pl
from jax.experimental.pallas import tpu as pltpu

B = 16
K = 5
V = 8192
H = 1024
E = 512
EE = 64
NEMO = 7
T = 8
SOS = 1
EOS = 2

RES = 1024
SW = 1024
S = (V - RES) // SW
BK = B * K
FLOOR = -1.0e30
FLOOR2 = -2.0e30
BIGIDX = 2 ** 30

F32 = jnp.float32


def _decode_kernel(enc_ref, emoid_ref, tok_ref, emoemb_ref, wih_ref,
                   whh_ref, bih_ref, bhh_ref, woutres_ref, boutres_ref,
                   wout_ref, bout_ref,
                   logp_ref, hid_out_ref, vocab_ref, beam_ref, score_ref,
                   hid_s, hnew_s, seq_s, ids_s, xcat_s, cvals_s, cidx_s,
                   selbeam_s, selvocab_s, selscore_s):
    t = pl.program_id(0)
    s = pl.program_id(1)

    @pl.when((t == 0) & (s == 0))
    def _init():
        enc = enc_ref[...]
        for k in range(K):
            hid_s[16 * k:16 * (k + 1), :] = enc
        seq_s[0:16, :] = jnp.zeros((16, 1), F32)
        seq_s[16:BK, :] = jnp.full((BK - 16, 1), -jnp.inf, F32)
        ids_s[...] = jnp.full((BK, 1), SOS, jnp.int32)
        lane8 = jax.lax.broadcasted_iota(jnp.int32, (BK, 8), 1)
        emo_oh = (lane8 == emoid_ref[...]).astype(F32)
        xcat_s[:, E:] = jnp.dot(emo_oh, emoemb_ref[...],
                                preferred_element_type=F32,
                                precision=jax.lax.Precision.HIGHEST)

    @pl.when(s == 0)
    def _gru():
        lanes_v = jax.lax.broadcasted_iota(jnp.int32, (BK, V), 1)
        onehot = (lanes_v == ids_s[...]).astype(F32)
        xcat_s[:, :E] = jnp.dot(onehot, tok_ref[...],
                                preferred_element_type=F32,
                                precision=jax.lax.Precision.HIGHEST)
        h = hid_s[...]
        gx = (jnp.dot(xcat_s[...], wih_ref[...], preferred_element_type=F32)
              + bih_ref[...])
        gh = (jnp.dot(h, whh_ref[...], preferred_element_type=F32)
              + bhh_ref[...])
        r = jax.nn.sigmoid(gx[:, :H] + gh[:, :H])
        z = jax.nn.sigmoid(gx[:, H:2 * H] + gh[:, H:2 * H])
        n = jnp.tanh(gx[:, 2 * H:] + r * gh[:, 2 * H:])
        hnew_s[...] = (1.0 - z) * n + z * h
        logp_ref[0, :, 0:RES] = (
            jnp.dot(hnew_s[...], woutres_ref[...],
                    preferred_element_type=F32) + boutres_ref[...])

    logp_ref[0, :, pl.ds(RES + s * SW, SW)] = (
        jnp.dot(hnew_s[...], wout_ref[...], preferred_element_type=F32)
        + bout_ref[...])

    @pl.when(s == S - 1)
    def _step():
        m = jnp.max(logp_ref[0, :, :], axis=1, keepdims=True)
        shifted = logp_ref[0, :, :] - m
        e = jnp.exp(shifted)
        parts = [e[:, c * 128:(c + 1) * 128] for c in range(V // 128)]
        while len(parts) > 1:
            parts = [parts[2 * i] + parts[2 * i + 1]
                     for i in range(len(parts) // 2)]
        acc = parts[0]
        w0 = 128
        while w0 > 1:
            w0 //= 2
            acc = acc[:, :w0] + acc[:, w0:2 * w0]
        logp_ref[0, :, :] = shifted - jnp.log(acc)

        cvals_s[...] = jnp.full((B, 32), FLOOR2, F32)
        cidx_s[...] = jnp.full((B, 32), BIGIDX, jnp.int32)
        lanes16 = jax.lax.broadcasted_iota(jnp.int32, (B, V), 1)
        for k in range(K):
            w = jnp.maximum(seq_s[16 * k:16 * (k + 1), :]
                            + logp_ref[0, 16 * k:16 * (k + 1), :], FLOOR)
            for j in range(K):
                mv = jnp.max(w, axis=1, keepdims=True)
                c = jnp.min(jnp.where(w == mv, lanes16, V),
                            axis=1, keepdims=True)
                col = k * K + j
                cvals_s[:, col:col + 1] = mv
                cidx_s[:, col:col + 1] = k * V + c
                w = jnp.where(lanes16 == c, FLOOR2, w)

        vals = cvals_s[...]
        idxs = cidx_s[...]
        vocab_ref[0, :, :] = jnp.zeros((B, 8), jnp.int32)
        beam_ref[0, :, :] = jnp.zeros((B, 8), jnp.int32)
        score_ref[0, :, :] = jnp.zeros((B, 8), F32)
        for j in range(K):
            mv = jnp.max(vals, axis=1, keepdims=True)
            fsel = jnp.min(jnp.where(vals == mv, idxs, BIGIDX),
                           axis=1, keepdims=True)
            vj = jnp.where(mv == FLOOR, -jnp.inf, mv)
            vocab = jnp.bitwise_and(fsel, V - 1)
            beamk = jnp.right_shift(fsel, 13)
            selvocab_s[16 * j:16 * (j + 1), :] = vocab
            selbeam_s[16 * j:16 * (j + 1), :] = beamk
            selscore_s[16 * j:16 * (j + 1), :] = vj
            vocab_ref[0, :, j:j + 1] = vocab
            beam_ref[0, :, j:j + 1] = beamk
            score_ref[0, :, j:j + 1] = vj
            vals = jnp.where(idxs == fsel, FLOOR2, vals)

        hn = [hnew_s[16 * k:16 * (k + 1), :] for k in range(K)]
        for j in range(K):
            bsel = selbeam_s[16 * j:16 * (j + 1), :]
            blk = hn[K - 1]
            for k in range(K - 2, -1, -1):
                blk = jnp.where(bsel == k, hn[k], blk)
            hid_s[16 * j:16 * (j + 1), :] = blk
            hid_out_ref[16 * j:16 * (j + 1), :] = blk

        newids = selvocab_s[...]
        seq_s[...] = jnp.where(newids == EOS, -jnp.inf, selscore_s[...])
        ids_s[...] = newids


@jax.jit
def kernel(encoder_hidden, emotion_inputs, tok_embed, emo_embed, W_ih, W_hh,
           b_ih, b_hh, W_out, b_out):
    enc = encoder_hidden[0]
    emoid = jnp.tile(emotion_inputs[:, 0].astype(jnp.int32), K)[:, None]
    emoemb8 = jnp.zeros((8, EE), F32).at[:NEMO].set(emo_embed)
    grid = (T, S)
    const2d = lambda t, s: (0, 0)
    out_shapes = (
        jax.ShapeDtypeStruct((T, BK, V), F32),
        jax.ShapeDtypeStruct((BK, H), F32),
        jax.ShapeDtypeStruct((T, B, 8), jnp.int32),
        jax.ShapeDtypeStruct((T, B, 8), jnp.int32),
        jax.ShapeDtypeStruct((T, B, 8), F32),
    )
    out_specs = (
        pl.BlockSpec((1, BK, V), lambda t, s: (t, 0, 0)),
        pl.BlockSpec((BK, H), const2d),
        pl.BlockSpec((1, B, 8), lambda t, s: (t, 0, 0)),
        pl.BlockSpec((1, B, 8), lambda t, s: (t, 0, 0)),
        pl.BlockSpec((1, B, 8), lambda t, s: (t, 0, 0)),
    )
    in_specs = [
        pl.BlockSpec((B, H), const2d),
        pl.BlockSpec((BK, 1), const2d),
        pl.BlockSpec((V, E), const2d),
        pl.BlockSpec((8, EE), const2d),
        pl.BlockSpec((E + EE, 3 * H), const2d),
        pl.BlockSpec((H, 3 * H), const2d),
        pl.BlockSpec((1, 3 * H), const2d),
        pl.BlockSpec((1, 3 * H), const2d),
        pl.BlockSpec((H, RES), const2d),
        pl.BlockSpec((1, RES), const2d),
        pl.BlockSpec((H, SW), lambda t, s: (0, s + RES // SW)),
        pl.BlockSpec((1, SW), lambda t, s: (0, s + RES // SW)),
    ]
    scratch = [
        pltpu.VMEM((BK, H), F32),
        pltpu.VMEM((BK, H), F32),
        pltpu.VMEM((BK, 1), F32),
        pltpu.VMEM((BK, 1), jnp.int32),
        pltpu.VMEM((BK, E + EE), F32),
        pltpu.VMEM((B, 32), F32),
        pltpu.VMEM((B, 32), jnp.int32),
        pltpu.VMEM((BK, 1), jnp.int32),
        pltpu.VMEM((BK, 1), jnp.int32),
        pltpu.VMEM((BK, 1), F32),
    ]

    logp_bm, hid_bm, vocab5, beam5, score5 = pl.pallas_call(
        _decode_kernel,
        grid=grid,
        in_specs=in_specs,
        out_specs=out_specs,
        out_shape=out_shapes,
        scratch_shapes=scratch,
    )(enc, emoid, tok_embed, emoemb8, W_ih, W_hh,
      b_ih[None, :], b_hh[None, :], W_out[:, :RES], b_out[None, :RES],
      W_out, b_out[None, :])

    logp = logp_bm.reshape(T, K, B, V).transpose(0, 2, 1, 3).reshape(T, BK, V)
    hidden = hid_bm.reshape(K, B, H).transpose(1, 0, 2).reshape(BK, H)[None]
    symbols = vocab5[:, :, :K].reshape(T, BK)
    preds = (beam5[:, :, :K] +
             (jnp.arange(B, dtype=jnp.int32) * K)[None, :, None]
             ).reshape(T, BK)
    scores = score5[:, :, :K].reshape(T, BK)
    return logp, hidden, symbols, preds, scores


# speedup vs baseline: 1.0845x; 1.0062x over previous
import jax
import jax.numpy as jnp
from jax.experimental import pallas as pl
from jax.experimental.pallas import tpu as pltpu

B = 16
K = 5
V = 8192
H = 1024
E = 512
EE = 64
NEMO = 7
T = 8
SOS = 1
EOS = 2

S = 8
VS = V // S
BK = B * K
FLOOR = -1.0e30
FLOOR2 = -2.0e30
BIGIDX = 2 ** 30

F32 = jnp.float32


def _decode_kernel(enc_ref, emoid_ref, tok_ref, emoemb_ref, wih_ref,
                   whh_ref, bih_ref, bhh_ref, wout_ref, bout_ref,
                   logp_ref, hid_out_ref, vocab_ref, beam_ref, score_ref,
                   hid_s, hnew_s, seq_s, ids_s, xcat_s, cvals_s, cidx_s,
                   selbeam_s, selvocab_s, selscore_s):
    t = pl.program_id(0)
    s = pl.program_id(1)

    @pl.when((t == 0) & (s == 0))
    def _init():
        enc = enc_ref[...]
        for k in range(K):
            hid_s[16 * k:16 * (k + 1), :] = enc
        seq_s[0:16, :] = jnp.zeros((16, 1), F32)
        seq_s[16:BK, :] = jnp.full((BK - 16, 1), -jnp.inf, F32)
        ids_s[...] = jnp.full((BK, 1), SOS, jnp.int32)
        lane8 = jax.lax.broadcasted_iota(jnp.int32, (BK, 8), 1)
        emo_oh = (lane8 == emoid_ref[...]).astype(F32)
        xcat_s[:, E:] = jnp.dot(emo_oh, emoemb_ref[...],
                                preferred_element_type=F32,
                                precision=jax.lax.Precision.HIGHEST)

    @pl.when(s == 0)
    def _gru():
        lanes_v = jax.lax.broadcasted_iota(jnp.int32, (BK, V), 1)
        onehot = (lanes_v == ids_s[...]).astype(F32)
        xcat_s[:, :E] = jnp.dot(onehot, tok_ref[...],
                                preferred_element_type=F32,
                                precision=jax.lax.Precision.HIGHEST)
        h = hid_s[...]
        gx = (jnp.dot(xcat_s[...], wih_ref[...], preferred_element_type=F32)
              + bih_ref[...])
        gh = (jnp.dot(h, whh_ref[...], preferred_element_type=F32)
              + bhh_ref[...])
        r = jax.nn.sigmoid(gx[:, :H] + gh[:, :H])
        z = jax.nn.sigmoid(gx[:, H:2 * H] + gh[:, H:2 * H])
        n = jnp.tanh(gx[:, 2 * H:] + r * gh[:, 2 * H:])
        hnew_s[...] = (1.0 - z) * n + z * h

    logp_ref[0, :, pl.ds(s * VS, VS)] = (
        jnp.dot(hnew_s[...], wout_ref[...], preferred_element_type=F32)
        + bout_ref[...])

    @pl.when(s == S - 1)
    def _step():
        m = jnp.max(logp_ref[0, :, :], axis=1, keepdims=True)
        shifted = logp_ref[0, :, :] - m
        e = jnp.exp(shifted)
        parts = [e[:, c * 128:(c + 1) * 128] for c in range(V // 128)]
        while len(parts) > 1:
            parts = [parts[2 * i] + parts[2 * i + 1]
                     for i in range(len(parts) // 2)]
        acc = parts[0]
        w0 = 128
        while w0 > 1:
            w0 //= 2
            acc = acc[:, :w0] + acc[:, w0:2 * w0]
        logp_ref[0, :, :] = shifted - jnp.log(acc)

        cvals_s[...] = jnp.full((B, 32), FLOOR2, F32)
        cidx_s[...] = jnp.full((B, 32), BIGIDX, jnp.int32)
        lanes16 = jax.lax.broadcasted_iota(jnp.int32, (B, V), 1)
        for k in range(K):
            w = jnp.maximum(seq_s[16 * k:16 * (k + 1), :]
                            + logp_ref[0, 16 * k:16 * (k + 1), :], FLOOR)
            for j in range(K):
                mv = jnp.max(w, axis=1, keepdims=True)
                c = jnp.min(jnp.where(w == mv, lanes16, V),
                            axis=1, keepdims=True)
                col = k * K + j
                cvals_s[:, col:col + 1] = mv
                cidx_s[:, col:col + 1] = k * V + c
                w = jnp.where(lanes16 == c, FLOOR2, w)

        vals = cvals_s[...]
        idxs = cidx_s[...]
        vocab_ref[0, :, :] = jnp.zeros((B, 8), jnp.int32)
        beam_ref[0, :, :] = jnp.zeros((B, 8), jnp.int32)
        score_ref[0, :, :] = jnp.zeros((B, 8), F32)
        for j in range(K):
            mv = jnp.max(vals, axis=1, keepdims=True)
            fsel = jnp.min(jnp.where(vals == mv, idxs, BIGIDX),
                           axis=1, keepdims=True)
            vj = jnp.where(mv == FLOOR, -jnp.inf, mv)
            vocab = jnp.bitwise_and(fsel, V - 1)
            beamk = jnp.right_shift(fsel, 13)
            selvocab_s[16 * j:16 * (j + 1), :] = vocab
            selbeam_s[16 * j:16 * (j + 1), :] = beamk
            selscore_s[16 * j:16 * (j + 1), :] = vj
            vocab_ref[0, :, j:j + 1] = vocab
            beam_ref[0, :, j:j + 1] = beamk
            score_ref[0, :, j:j + 1] = vj
            vals = jnp.where(idxs == fsel, FLOOR2, vals)

        hn = [hnew_s[16 * k:16 * (k + 1), :] for k in range(K)]
        for j in range(K):
            bsel = selbeam_s[16 * j:16 * (j + 1), :]
            blk = hn[K - 1]
            for k in range(K - 2, -1, -1):
                blk = jnp.where(bsel == k, hn[k], blk)
            hid_s[16 * j:16 * (j + 1), :] = blk
            hid_out_ref[16 * j:16 * (j + 1), :] = blk

        newids = selvocab_s[...]
        seq_s[...] = jnp.where(newids == EOS, -jnp.inf, selscore_s[...])
        ids_s[...] = newids


@jax.jit
def kernel(encoder_hidden, emotion_inputs, tok_embed, emo_embed, W_ih, W_hh,
           b_ih, b_hh, W_out, b_out):
    enc = encoder_hidden[0]
    emoid = jnp.tile(emotion_inputs[:, 0].astype(jnp.int32), K)[:, None]
    emoemb8 = jnp.zeros((8, EE), F32).at[:NEMO].set(emo_embed)
    grid = (T, S)
    const2d = lambda t, s: (0, 0)
    out_shapes = (
        jax.ShapeDtypeStruct((T, BK, V), F32),
        jax.ShapeDtypeStruct((BK, H), F32),
        jax.ShapeDtypeStruct((T, B, 8), jnp.int32),
        jax.ShapeDtypeStruct((T, B, 8), jnp.int32),
        jax.ShapeDtypeStruct((T, B, 8), F32),
    )
    out_specs = (
        pl.BlockSpec((1, BK, V), lambda t, s: (t, 0, 0)),
        pl.BlockSpec((BK, H), const2d),
        pl.BlockSpec((1, B, 8), lambda t, s: (t, 0, 0)),
        pl.BlockSpec((1, B, 8), lambda t, s: (t, 0, 0)),
        pl.BlockSpec((1, B, 8), lambda t, s: (t, 0, 0)),
    )
    in_specs = [
        pl.BlockSpec((B, H), const2d),
        pl.BlockSpec((BK, 1), const2d),
        pl.BlockSpec((V, E), const2d),
        pl.BlockSpec((8, EE), const2d),
        pl.BlockSpec((E + EE, 3 * H), const2d),
        pl.BlockSpec((H, 3 * H), const2d),
        pl.BlockSpec((1, 3 * H), const2d),
        pl.BlockSpec((1, 3 * H), const2d),
        pl.BlockSpec((H, VS), lambda t, s: (0, s)),
        pl.BlockSpec((1, VS), lambda t, s: (0, s)),
    ]
    scratch = [
        pltpu.VMEM((BK, H), F32),
        pltpu.VMEM((BK, H), F32),
        pltpu.VMEM((BK, 1), F32),
        pltpu.VMEM((BK, 1), jnp.int32),
        pltpu.VMEM((BK, E + EE), F32),
        pltpu.VMEM((B, 32), F32),
        pltpu.VMEM((B, 32), jnp.int32),
        pltpu.VMEM((BK, 1), jnp.int32),
        pltpu.VMEM((BK, 1), jnp.int32),
        pltpu.VMEM((BK, 1), F32),
    ]

    logp_bm, hid_bm, vocab5, beam5, score5 = pl.pallas_call(
        _decode_kernel,
        grid=grid,
        in_specs=in_specs,
        out_specs=out_specs,
        out_shape=out_shapes,
        scratch_shapes=scratch,
    )(enc, emoid, tok_embed, emoemb8, W_ih, W_hh,
      b_ih[None, :], b_hh[None, :], W_out, b_out[None, :])

    logp = logp_bm.reshape(T, K, B, V).transpose(0, 2, 1, 3).reshape(T, BK, V)
    hidden = hid_bm.reshape(K, B, H).transpose(1, 0, 2).reshape(BK, H)[None]
    symbols = vocab5[:, :, :K].reshape(T, BK)
    preds = (beam5[:, :, :K] +
             (jnp.arange(B, dtype=jnp.int32) * K)[None, :, None]
             ).reshape(T, BK)
    scores = score5[:, :, :K].reshape(T, BK)
    return logp, hidden, symbols, preds, scores
